# baseline (speedup 1.0000x reference)
import functools

import jax
import jax.numpy as jnp
from jax import lax
from jax.experimental import pallas as pl
from jax.experimental.pallas import tpu as pltpu
from jax.experimental.pallas import tpu_sc as plsc

_CROP = 14
_EPS = 1e-7
_C = 128
_N = 1000
_M = _N * _CROP * _CROP
_K = 56
_KPAD = 64
_NCHUNK = _M // _K
_NW = 32
_NLOOP = (_NCHUNK + _NW - 1) // _NW
_LEVEL_DIM = (256, 128, 64, 32, 16)
_LEVEL_BASE = (0, 65536, 81920, 86016, 87040)
_P = 87296


def _host_plan(image_shape, boxes):
    img = image_shape.astype(jnp.float32)
    b = boxes[0]
    bx1 = b[:, 0]
    by1 = b[:, 1]
    bx2 = b[:, 2]
    by2 = b[:, 3]
    w = bx2 - bx1
    h = by2 - by1
    size = jnp.sqrt(w * h)
    log = jnp.log(size / 224.0 + _EPS)
    log2 = log / jnp.log(jnp.asarray(2.0, log.dtype))
    levels = jnp.clip(jnp.floor(1.0 + log2), 0.0, 4.0)

    iy = jnp.arange(_CROP, dtype=jnp.float32)
    idx_tl = idx_tr = idx_bl = idx_br = None
    mask = xl = yl = None
    for lvl, dim in enumerate(_LEVEL_DIM):
        fh = jnp.float32(dim)
        fw = jnp.float32(dim)
        H = W = dim
        ny1 = by1 / img[1] * fh / (fh - 1)
        nx1 = bx1 / img[2] * fw / (fw - 1)
        ny2 = (by2 / img[1] * fh - 1) / (fh - 1)
        nx2 = (bx2 / img[2] * fw - 1) / (fw - 1)
        in_y = ny1[:, None] * (H - 1) + iy[None, :] * (
            (ny2 - ny1)[:, None] * (H - 1) / (_CROP - 1))
        in_x = nx1[:, None] * (W - 1) + iy[None, :] * (
            (nx2 - nx1)[:, None] * (W - 1) / (_CROP - 1))
        valid_y = (in_y >= 0) & (in_y <= H - 1)
        valid_x = (in_x >= 0) & (in_x <= W - 1)
        top = jnp.floor(in_y)
        left = jnp.floor(in_x)
        y_lerp = in_y - top
        x_lerp = in_x - left
        y0i = jnp.clip(top, 0, H - 1).astype(jnp.int32)
        y1i = jnp.clip(top + 1, 0, H - 1).astype(jnp.int32)
        x0i = jnp.clip(left, 0, W - 1).astype(jnp.int32)
        x1i = jnp.clip(left + 1, 0, W - 1).astype(jnp.int32)

        row0 = _LEVEL_BASE[lvl] + y0i * W
        row1 = _LEVEL_BASE[lvl] + y1i * W
        l_tl = row0[:, :, None] + x0i[:, None, :]
        l_tr = row0[:, :, None] + x1i[:, None, :]
        l_bl = row1[:, :, None] + x0i[:, None, :]
        l_br = row1[:, :, None] + x1i[:, None, :]
        l_mask = (valid_y[:, :, None] & valid_x[:, None, :]).astype(jnp.float32)
        l_xl = jnp.broadcast_to(x_lerp[:, None, :], (_N, _CROP, _CROP))
        l_yl = jnp.broadcast_to(y_lerp[:, :, None], (_N, _CROP, _CROP))

        sel = (levels == float(lvl))[:, None, None]
        if idx_tl is None:
            idx_tl, idx_tr, idx_bl, idx_br = l_tl, l_tr, l_bl, l_br
            mask, xl, yl = l_mask, l_xl, l_yl
        else:
            idx_tl = jnp.where(sel, l_tl, idx_tl)
            idx_tr = jnp.where(sel, l_tr, idx_tr)
            idx_bl = jnp.where(sel, l_bl, idx_bl)
            idx_br = jnp.where(sel, l_br, idx_br)
            mask = jnp.where(sel, l_mask, mask)
            xl = jnp.where(sel, l_xl, xl)
            yl = jnp.where(sel, l_yl, yl)

    idx = jnp.stack([idx_tl, idx_tr, idx_bl, idx_br], 0)
    idx = idx.transpose(0, 2, 3, 1)
    idx = idx.reshape(4, _NCHUNK, _K).transpose(1, 0, 2)
    idx = jnp.pad(idx, ((0, _NLOOP * _NW - _NCHUNK), (0, 0), (0, 0)))
    idx = idx.reshape(_NLOOP, _NW, 4, _K).transpose(1, 0, 2, 3)
    wts = jnp.stack([xl, yl, mask], 0).transpose(0, 2, 3, 1)
    wts = wts.reshape(3, _NCHUNK, _K).transpose(1, 0, 2)
    wts = jnp.pad(wts, ((0, _NLOOP * _NW - _NCHUNK), (0, 0), (0, _KPAD - _K)))
    wts = wts.reshape(_NLOOP, _NW, 3, _KPAD).transpose(1, 0, 2, 3)
    return idx, wts


_GD_NUMS = lax.GatherDimensionNumbers(
    offset_dims=(), collapsed_slice_dims=(0,), start_index_map=(0,))


def _bcast_lane(vec, lane_splat):
    return lax.gather(vec, lane_splat[:, None], _GD_NUMS, (1,),
                      mode=lax.GatherScatterMode.PROMISE_IN_BOUNDS)


def _roi_body(table, idx_hbm, w_hbm, out_hbm,
              idx_all, w_v0, w_v1, rows_v0, rows_v1, out_v,
              sem0, sem1, semw0, semw1):
    cid = lax.axis_index("c")
    sid = lax.axis_index("s")
    wid = sid * 2 + cid
    nloop = (_NCHUNK + _NW - 1) // _NW
    bufs = ((rows_v0, sem0, w_v0, semw0), (rows_v1, sem1, w_v1, semw1))

    pltpu.sync_copy(idx_hbm.at[wid], idx_all)

    def fire(k, buf):
        rows_v, sem, w_v, semw = buf
        for c in range(4):
            pltpu.async_copy(table.at[idx_all.at[k, c]], rows_v.at[c], sem)
        pltpu.async_copy(w_hbm.at[wid, k], w_v, semw)

    def process(k, j, buf):
        rows_v, sem, w_v, semw = buf
        for c in range(4):
            pltpu.make_async_copy(table.at[idx_all.at[k, c]], rows_v.at[c],
                                  sem).wait()
        pltpu.make_async_copy(w_hbm.at[wid, k], w_v, semw).wait()

        for g2 in range(0, _K, 16):
            cnt = min(16, _K - g2)
            wx = w_v[0, pl.ds(g2, 16)]
            wy = w_v[1, pl.ds(g2, 16)]
            wm = w_v[2, pl.ds(g2, 16)]

            def pix(i, c2, g2=g2, wx=wx, wy=wy, wm=wm):
                p = g2 + i
                lane = jnp.full((16,), i, jnp.int32)
                xlv = _bcast_lane(wx, lane)
                ylv = _bcast_lane(wy, lane)
                mkv = _bcast_lane(wm, lane)
                for g in range(_C // 16):
                    sl = pl.ds(g * 16, 16)
                    tl = rows_v[0, p, sl]
                    tr = rows_v[1, p, sl]
                    bl = rows_v[2, p, sl]
                    br = rows_v[3, p, sl]
                    top_i = tl + (tr - tl) * xlv
                    bot_i = bl + (br - bl) * xlv
                    out_v[p, sl] = (top_i + (bot_i - top_i) * ylv) * mkv
                return c2

            lax.fori_loop(0, cnt, pix, 0)

        pltpu.sync_copy(out_v, out_hbm.at[pl.ds(j * _K, _K)])

    fire(0, bufs[0])

    def outer(k2, carry):
        for b in (0, 1):
            k = k2 * 2 + b
            j = wid + k * _NW
            jn = wid + (k + 1) * _NW

            @pl.when(jn < _NCHUNK)
            def _():
                fire(k + 1, bufs[1 - b])

            @pl.when(j < _NCHUNK)
            def _():
                process(k, j, bufs[b])

        return carry

    lax.fori_loop(0, nloop // 2, outer, 0)


def _concat_body(f0, f1, f2, f3, f4, out, sem):
    copies = []
    for lvl, f in enumerate((f0, f1, f2, f3, f4)):
        rows = _LEVEL_DIM[lvl] * _LEVEL_DIM[lvl]
        cp = pltpu.make_async_copy(f, out.at[pl.ds(_LEVEL_BASE[lvl], rows)], sem)
        cp.start()
        copies.append(cp)
    for cp in copies:
        cp.wait()


def _concat_table(fpns):
    return pl.pallas_call(
        _concat_body,
        in_specs=[pl.BlockSpec(memory_space=pltpu.MemorySpace.HBM)] * 5,
        out_specs=pl.BlockSpec(memory_space=pltpu.MemorySpace.HBM),
        out_shape=jax.ShapeDtypeStruct((_P, _C), jnp.float32),
        scratch_shapes=[pltpu.SemaphoreType.DMA],
    )(*[f[0].reshape(-1, _C) for f in fpns])


@functools.lru_cache(maxsize=1)
def _build_roi_gather():
    return pl.kernel(
        _roi_body,
        out_type=jax.ShapeDtypeStruct((_M, _C), jnp.float32),
        mesh=plsc.VectorSubcoreMesh(core_axis_name="c", subcore_axis_name="s"),
        scratch_types=[
            pltpu.VMEM((_NLOOP, 4, _K), jnp.int32),
            pltpu.VMEM((3, _KPAD), jnp.float32),
            pltpu.VMEM((3, _KPAD), jnp.float32),
            pltpu.VMEM((4, _K, _C), jnp.float32),
            pltpu.VMEM((4, _K, _C), jnp.float32),
            pltpu.VMEM((_K, _C), jnp.float32),
            pltpu.SemaphoreType.DMA,
            pltpu.SemaphoreType.DMA,
            pltpu.SemaphoreType.DMA,
            pltpu.SemaphoreType.DMA,
        ],
    )


def kernel(image_shape, boxes, scores, fpn0, fpn1, fpn2, fpn3, fpn4):
    del scores
    boxes = jax.lax.stop_gradient(boxes)
    fpns = [jax.lax.stop_gradient(f) for f in (fpn0, fpn1, fpn2, fpn3, fpn4)]
    table = _concat_table(fpns)
    idx, wts = _host_plan(image_shape, boxes)
    out = _build_roi_gather()(table, idx, wts)
    out = out.reshape(1, _CROP, _CROP, _N, _C)
    return out.transpose(0, 3, 1, 2, 4)


# speedup vs baseline: 5.4016x; 5.4016x over previous
import functools

import jax
import jax.numpy as jnp
from jax import lax
from jax.experimental import pallas as pl
from jax.experimental.pallas import tpu as pltpu
from jax.experimental.pallas import tpu_sc as plsc

_CROP = 14
_EPS = 1e-7
_C = 128
_N = 1000
_M = _N * _CROP * _CROP
_K = 56
_KPAD = 64
_NCHUNK = _M // _K
_NW = 32
_NLOOP = (_NCHUNK + _NW - 1) // _NW
_LEVEL_DIM = (256, 128, 64, 32, 16)
_LEVEL_BASE = (0, 65536, 81920, 86016, 87040)
_P = 87296


def _host_plan(image_shape, boxes):
    img = image_shape.astype(jnp.float32)
    b = boxes[0]
    bx1 = b[:, 0]
    by1 = b[:, 1]
    bx2 = b[:, 2]
    by2 = b[:, 3]
    w = bx2 - bx1
    h = by2 - by1
    size = jnp.sqrt(w * h)
    log = jnp.log(size / 224.0 + _EPS)
    log2 = log / jnp.log(jnp.asarray(2.0, log.dtype))
    levels = jnp.clip(jnp.floor(1.0 + log2), 0.0, 4.0)

    iy = jnp.arange(_CROP, dtype=jnp.float32)
    idx_tl = idx_tr = idx_bl = idx_br = None
    mask = xl = yl = None
    for lvl, dim in enumerate(_LEVEL_DIM):
        fh = jnp.float32(dim)
        fw = jnp.float32(dim)
        H = W = dim
        ny1 = by1 / img[1] * fh / (fh - 1)
        nx1 = bx1 / img[2] * fw / (fw - 1)
        ny2 = (by2 / img[1] * fh - 1) / (fh - 1)
        nx2 = (bx2 / img[2] * fw - 1) / (fw - 1)
        in_y = ny1[:, None] * (H - 1) + iy[None, :] * (
            (ny2 - ny1)[:, None] * (H - 1) / (_CROP - 1))
        in_x = nx1[:, None] * (W - 1) + iy[None, :] * (
            (nx2 - nx1)[:, None] * (W - 1) / (_CROP - 1))
        valid_y = (in_y >= 0) & (in_y <= H - 1)
        valid_x = (in_x >= 0) & (in_x <= W - 1)
        top = jnp.floor(in_y)
        left = jnp.floor(in_x)
        y_lerp = in_y - top
        x_lerp = in_x - left
        y0i = jnp.clip(top, 0, H - 1).astype(jnp.int32)
        y1i = jnp.clip(top + 1, 0, H - 1).astype(jnp.int32)
        x0i = jnp.clip(left, 0, W - 1).astype(jnp.int32)
        x1i = jnp.clip(left + 1, 0, W - 1).astype(jnp.int32)

        row0 = _LEVEL_BASE[lvl] + y0i * W
        row1 = _LEVEL_BASE[lvl] + y1i * W
        l_tl = row0[:, :, None] + x0i[:, None, :]
        l_tr = row0[:, :, None] + x1i[:, None, :]
        l_bl = row1[:, :, None] + x0i[:, None, :]
        l_br = row1[:, :, None] + x1i[:, None, :]
        l_mask = (valid_y[:, :, None] & valid_x[:, None, :]).astype(jnp.float32)
        l_xl = jnp.broadcast_to(x_lerp[:, None, :], (_N, _CROP, _CROP))
        l_yl = jnp.broadcast_to(y_lerp[:, :, None], (_N, _CROP, _CROP))

        sel = (levels == float(lvl))[:, None, None]
        if idx_tl is None:
            idx_tl, idx_tr, idx_bl, idx_br = l_tl, l_tr, l_bl, l_br
            mask, xl, yl = l_mask, l_xl, l_yl
        else:
            idx_tl = jnp.where(sel, l_tl, idx_tl)
            idx_tr = jnp.where(sel, l_tr, idx_tr)
            idx_bl = jnp.where(sel, l_bl, idx_bl)
            idx_br = jnp.where(sel, l_br, idx_br)
            mask = jnp.where(sel, l_mask, mask)
            xl = jnp.where(sel, l_xl, xl)
            yl = jnp.where(sel, l_yl, yl)

    idx = jnp.stack([idx_tl, idx_tr, idx_bl, idx_br], 0)
    idx = idx.transpose(0, 2, 3, 1)
    idx = idx.reshape(4, _NCHUNK, _K).transpose(1, 0, 2)
    idx = jnp.pad(idx, ((0, _NLOOP * _NW - _NCHUNK), (0, 0), (0, 0)))
    idx = idx.reshape(_NLOOP, _NW, 4, _K).transpose(1, 0, 2, 3)
    wts = jnp.stack([xl, yl, mask], 0).transpose(0, 2, 3, 1)
    wts = wts.reshape(3, _NCHUNK, _K).transpose(1, 0, 2)
    wts = jnp.pad(wts, ((0, _NLOOP * _NW - _NCHUNK), (0, 0), (0, _KPAD - _K)))
    wts = wts.reshape(_NLOOP, _NW, 3, _KPAD).transpose(1, 0, 2, 3)
    return idx, wts


_GD_NUMS = lax.GatherDimensionNumbers(
    offset_dims=(), collapsed_slice_dims=(0,), start_index_map=(0,))


def _bcast_lane(vec, lane_splat):
    return lax.gather(vec, lane_splat[:, None], _GD_NUMS, (1,),
                      mode=lax.GatherScatterMode.PROMISE_IN_BOUNDS)


def _roi_body(table, idx_hbm, w_hbm, out_hbm,
              idx_all, w_v0, w_v1, rows_v0, rows_v1, out_v,
              sem0, sem1, semw0, semw1):
    cid = lax.axis_index("c")
    sid = lax.axis_index("s")
    wid = sid * 2 + cid
    nloop = (_NCHUNK + _NW - 1) // _NW
    bufs = ((rows_v0, sem0, w_v0, semw0), (rows_v1, sem1, w_v1, semw1))

    pltpu.sync_copy(idx_hbm.at[wid], idx_all)

    def fire(k, buf):
        rows_v, sem, w_v, semw = buf
        for c in range(4):
            pltpu.async_copy(table.at[idx_all.at[k, c]], rows_v.at[c], sem)
        pltpu.async_copy(w_hbm.at[wid, k], w_v, semw)

    def process(k, j, buf):
        rows_v, sem, w_v, semw = buf
        for c in range(4):
            pltpu.make_async_copy(table.at[idx_all.at[k, c]], rows_v.at[c],
                                  sem).wait()
        pltpu.make_async_copy(w_hbm.at[wid, k], w_v, semw).wait()

        for g2 in range(0, _K, 16):
            cnt = min(16, _K - g2)
            wx = w_v[0, pl.ds(g2, 16)]
            wy = w_v[1, pl.ds(g2, 16)]
            wm = w_v[2, pl.ds(g2, 16)]

            def pix(i, c2, g2=g2, wx=wx, wy=wy, wm=wm):
                p = g2 + i
                lane = jnp.full((16,), i, jnp.int32)
                xlv = _bcast_lane(wx, lane)
                ylv = _bcast_lane(wy, lane)
                mkv = _bcast_lane(wm, lane)
                for g in range(_C // 16):
                    sl = pl.ds(g * 16, 16)
                    tl = rows_v[0, p, sl]
                    tr = rows_v[1, p, sl]
                    bl = rows_v[2, p, sl]
                    br = rows_v[3, p, sl]
                    top_i = tl + (tr - tl) * xlv
                    bot_i = bl + (br - bl) * xlv
                    out_v[p, sl] = (top_i + (bot_i - top_i) * ylv) * mkv
                return c2

            lax.fori_loop(0, cnt, pix, 0)

        pltpu.sync_copy(out_v, out_hbm.at[pl.ds(j * _K, _K)])

    fire(0, bufs[0])

    def outer(k2, carry):
        for b in (0, 1):
            k = k2 * 2 + b
            j = wid + k * _NW
            jn = wid + (k + 1) * _NW

            @pl.when(jn < _NCHUNK)
            def _():
                fire(k + 1, bufs[1 - b])

            @pl.when(j < _NCHUNK)
            def _():
                process(k, j, bufs[b])

        return carry

    lax.fori_loop(0, nloop // 2, outer, 0)


@functools.lru_cache(maxsize=1)
def _build_roi_gather():
    return pl.kernel(
        _roi_body,
        out_type=jax.ShapeDtypeStruct((_M, _C), jnp.float32),
        mesh=plsc.VectorSubcoreMesh(core_axis_name="c", subcore_axis_name="s"),
        scratch_types=[
            pltpu.VMEM((_NLOOP, 4, _K), jnp.int32),
            pltpu.VMEM((3, _KPAD), jnp.float32),
            pltpu.VMEM((3, _KPAD), jnp.float32),
            pltpu.VMEM((4, _K, _C), jnp.float32),
            pltpu.VMEM((4, _K, _C), jnp.float32),
            pltpu.VMEM((_K, _C), jnp.float32),
            pltpu.SemaphoreType.DMA,
            pltpu.SemaphoreType.DMA,
            pltpu.SemaphoreType.DMA,
            pltpu.SemaphoreType.DMA,
        ],
    )


def kernel(image_shape, boxes, scores, fpn0, fpn1, fpn2, fpn3, fpn4):
    del scores
    boxes = jax.lax.stop_gradient(boxes)
    fpns = [jax.lax.stop_gradient(f) for f in (fpn0, fpn1, fpn2, fpn3, fpn4)]
    table = jnp.concatenate([f[0].reshape(-1, _C) for f in fpns], axis=0)
    idx, wts = _host_plan(image_shape, boxes)
    out = _build_roi_gather()(table, idx, wts)
    out = out.reshape(1, _CROP, _CROP, _N, _C)
    return out.transpose(0, 3, 1, 2, 4)
